# Initial kernel scaffold; baseline (speedup 1.0000x reference)
import jax
import jax.numpy as jnp
from jax.experimental import pallas as pl


def kernel(x, W1, b1, W2, b2, P1, pb1, P2, pb2, edge_index, v_idx, a_idx, b_idx):
    raise NotImplementedError("write your pallas kernel here")


# speedup vs baseline: 23.5424x; 23.5424x over previous
import functools

import jax
import jax.numpy as jnp
from jax import lax
from jax.experimental import pallas as pl
from jax.experimental.pallas import tpu as pltpu, tpu_sc as plsc

N = 10000
E = 320000
D = 128
H = 128
B = 4096
TAU = 0.2

NROW = 10240
ET = E + N
EROWS = 2816
EPAD = EROWS * 128 - ET
NCORE = 2
NSUB = 16
NW = NCORE * NSUB
ROWS_PER_TILE = NROW // NSUB
WBATCH = EROWS // NW
SBATCH = EROWS // NSUB
BLK = 2048
NG = NROW // BLK

_MESH = plsc.VectorSubcoreMesh(
    core_axis_name="c", subcore_axis_name="s", num_cores=NCORE,
    num_subcores=NSUB)
_SC_PARAMS = pltpu.CompilerParams(
    use_tc_tiling_on_sc=False, needs_layout_passes=False)


def _zero_vmem(ref, nrows, ncolgroups):
  z = jnp.zeros((16,), jnp.float32)

  def body(i, _):
    for k in range(ncolgroups):
      ref[i, pl.ds(k * 16, 16)] = z
    return None

  lax.fori_loop(0, nrows, body, None)


@functools.partial(
    pl.kernel,
    out_type=jax.ShapeDtypeStruct((NCORE, NROW, 16), jnp.float32),
    mesh=_MESH,
    compiler_params=_SC_PARAMS,
    scratch_types=[
        pltpu.VMEM((WBATCH, 128), jnp.int32),
        pltpu.VMEM((128, 16), jnp.float32),
        pltpu.VMEM((64, 16), jnp.float32),
        pltpu.VMEM_SHARED((NROW, 16), jnp.float32),
    ],
)
def _sc_deg(dst_hbm, out_hbm, idx_v, ones_v, zbuf, acc):
  c = lax.axis_index("c")
  s = lax.axis_index("s")
  w = s * NCORE + c
  rows0 = s * ROWS_PER_TILE

  _zero_vmem(zbuf, 64, 1)
  one = jnp.ones((16,), jnp.float32)

  def fill_ones(i, _):
    ones_v[i, :] = one
    return None

  lax.fori_loop(0, 128, fill_ones, None)
  for j in range(ROWS_PER_TILE // 64):
    pltpu.sync_copy(zbuf, acc.at[pl.ds(rows0 + j * 64, 64)])
  pltpu.sync_copy(dst_hbm.at[pl.ds(w * WBATCH, WBATCH)], idx_v)
  plsc.subcore_barrier()

  def body(j, _):
    pltpu.sync_copy(ones_v, acc.at[idx_v.at[j]], add=True)
    return None

  lax.fori_loop(0, WBATCH, body, None)
  plsc.subcore_barrier()
  for j in range(ROWS_PER_TILE // 64):
    pltpu.sync_copy(acc.at[pl.ds(rows0 + j * 64, 64)], zbuf)
    pltpu.sync_copy(zbuf, out_hbm.at[c, pl.ds(rows0 + j * 64, 64)])


def _mp_body(c, s, tbl_hbm, src2_hbm, dst_hbm, sidx, didx, rbuf0, rbuf1,
             zbuf, acc, sem0, sem1):
  rows0 = s * ROWS_PER_TILE
  _zero_vmem(zbuf, 64, 4)
  for j in range(ROWS_PER_TILE // 64):
    pltpu.sync_copy(zbuf, acc.at[pl.ds(rows0 + j * 64, 64)])
  pltpu.sync_copy(src2_hbm.at[c, pl.ds(s * SBATCH, SBATCH)], sidx)
  pltpu.sync_copy(dst_hbm.at[pl.ds(s * SBATCH, SBATCH)], didx)
  plsc.subcore_barrier()

  pltpu.make_async_copy(tbl_hbm.at[sidx.at[0]], rbuf0, sem0).start()
  pltpu.make_async_copy(tbl_hbm.at[sidx.at[1]], rbuf1, sem1).start()

  def body(jj, _):
    j0 = 2 * jj
    pltpu.make_async_copy(tbl_hbm.at[sidx.at[j0]], rbuf0, sem0).wait()
    pltpu.sync_copy(rbuf0, acc.at[didx.at[j0]], add=True)
    pltpu.make_async_copy(tbl_hbm.at[sidx.at[j0 + 2]], rbuf0, sem0).start()
    pltpu.make_async_copy(tbl_hbm.at[sidx.at[j0 + 1]], rbuf1, sem1).wait()
    pltpu.sync_copy(rbuf1, acc.at[didx.at[j0 + 1]], add=True)
    pltpu.make_async_copy(tbl_hbm.at[sidx.at[j0 + 3]], rbuf1, sem1).start()
    return None

  lax.fori_loop(0, SBATCH // 2 - 1, body, None)
  pltpu.make_async_copy(tbl_hbm.at[sidx.at[SBATCH - 2]], rbuf0, sem0).wait()
  pltpu.sync_copy(rbuf0, acc.at[didx.at[SBATCH - 2]], add=True)
  pltpu.make_async_copy(tbl_hbm.at[sidx.at[SBATCH - 1]], rbuf1, sem1).wait()
  pltpu.sync_copy(rbuf1, acc.at[didx.at[SBATCH - 1]], add=True)
  plsc.subcore_barrier()


_MP_SCRATCH = [
    pltpu.VMEM((SBATCH, 128), jnp.int32),
    pltpu.VMEM((SBATCH, 128), jnp.int32),
    pltpu.VMEM((128, 64), jnp.float32),
    pltpu.VMEM((128, 64), jnp.float32),
    pltpu.VMEM((64, 64), jnp.float32),
    pltpu.VMEM_SHARED((NROW, 64), jnp.float32),
    pltpu.SemaphoreType.DMA,
    pltpu.SemaphoreType.DMA,
]


@functools.partial(
    pl.kernel,
    out_type=jax.ShapeDtypeStruct((NCORE, NROW, 64), jnp.float32),
    mesh=_MESH,
    compiler_params=_SC_PARAMS,
    scratch_types=_MP_SCRATCH,
)
def _sc_mp(tbl_hbm, src2_hbm, dst_hbm, out_hbm, sidx, didx, rbuf0, rbuf1,
           zbuf, acc, sem0, sem1):
  c = lax.axis_index("c")
  s = lax.axis_index("s")
  _mp_body(c, s, tbl_hbm, src2_hbm, dst_hbm, sidx, didx, rbuf0, rbuf1,
           zbuf, acc, sem0, sem1)
  rows0 = s * ROWS_PER_TILE
  for j in range(ROWS_PER_TILE // 128):
    pltpu.sync_copy(acc.at[pl.ds(rows0 + j * 128, 128)], rbuf0)
    pltpu.sync_copy(rbuf0, out_hbm.at[c, pl.ds(rows0 + j * 128, 128)])


@functools.partial(
    pl.kernel,
    out_type=(
        jax.ShapeDtypeStruct((2 * NROW, 64), jnp.float32),
        jax.ShapeDtypeStruct((3, NCORE, B, 64), jnp.float32),
        jax.ShapeDtypeStruct((3, B, 16), jnp.float32),
    ),
    mesh=_MESH,
    compiler_params=_SC_PARAMS,
    scratch_types=_MP_SCRATCH + [
        pltpu.VMEM((3, 256), jnp.int32),
        pltpu.VMEM((128, 64), jnp.float32),
        pltpu.VMEM((256, 16), jnp.float32),
    ],
)
def _sc_mp2(tbl_hbm, src2_hbm, dst_hbm, vab2_hbm, dinv16_hbm,
            emb_hbm, trip_hbm, dg_hbm,
            sidx, didx, rbuf0, rbuf1, zbuf, acc, sem0, sem1,
            tidx, trbuf, dgbuf):
  c = lax.axis_index("c")
  s = lax.axis_index("s")
  for t in range(3):
    pltpu.sync_copy(vab2_hbm.at[c, t, 0, pl.ds(s * 256, 256)], tidx.at[t])

  _mp_body(c, s, tbl_hbm, src2_hbm, dst_hbm, sidx, didx, rbuf0, rbuf1,
           zbuf, acc, sem0, sem1)

  rows0 = s * ROWS_PER_TILE
  for j in range(ROWS_PER_TILE // 128):
    pltpu.sync_copy(acc.at[pl.ds(rows0 + j * 128, 128)], rbuf0)
    pltpu.sync_copy(rbuf0, emb_hbm.at[pl.ds(c * NROW + rows0 + j * 128, 128)])
  plsc.subcore_barrier()

  for t in range(3):
    for q in range(2):
      pltpu.async_copy(
          emb_hbm.at[tidx.at[t, pl.ds(q * 128, 128)]], trbuf, sem0).wait()
      pltpu.sync_copy(
          trbuf, trip_hbm.at[t, c, pl.ds(s * 256 + q * 128, 128)])

  @pl.when(c == 0)
  def _():
    for t in range(3):
      pltpu.async_copy(dinv16_hbm.at[tidx.at[t]], dgbuf, sem1).wait()
      pltpu.sync_copy(dgbuf, dg_hbm.at[t, pl.ds(s * 256, 256)])


def _dinv_block(degp_ref, i):
  deg = degp_ref[0, :, 0:1] + degp_ref[1, :, 0:1]
  dinv = lax.rsqrt(jnp.maximum(deg, 1.0))
  row = lax.broadcasted_iota(jnp.int32, (BLK, 1), 0) + i * BLK
  return jnp.where(row < N, dinv, 0.0)


def _tc1_body(x_ref, w_ref, b_ref, degp_ref, g_ref, dinv_ref):
  dinv = _dinv_block(degp_ref, pl.program_id(0))
  hw = jnp.dot(x_ref[...], w_ref[...],
               preferred_element_type=jnp.float32) + b_ref[...]
  g = hw * dinv
  g_ref[0] = g[:, :64]
  g_ref[1] = g[:, 64:]
  dinv_ref[...] = jnp.broadcast_to(dinv, (BLK, 16))


def _tc1(xp, w1, b1r, degp):
  return pl.pallas_call(
      _tc1_body,
      grid=(NG,),
      in_specs=[
          pl.BlockSpec((BLK, 128), lambda i: (i, 0)),
          pl.BlockSpec((128, 128), lambda i: (0, 0)),
          pl.BlockSpec((1, 128), lambda i: (0, 0)),
          pl.BlockSpec((2, BLK, 16), lambda i: (0, i, 0)),
      ],
      out_specs=[
          pl.BlockSpec((2, BLK, 64), lambda i: (0, i, 0)),
          pl.BlockSpec((BLK, 16), lambda i: (i, 0)),
      ],
      out_shape=[
          jax.ShapeDtypeStruct((2, NROW, 64), jnp.float32),
          jax.ShapeDtypeStruct((NROW, 16), jnp.float32),
      ],
  )(xp, w1, b1r, degp)


def _tc2_body(acc_ref, w_ref, b_ref, degp_ref, g_ref):
  dinv = _dinv_block(degp_ref, pl.program_id(0))
  h1a = jnp.maximum(acc_ref[0] * dinv, 0.0)
  h1b = jnp.maximum(acc_ref[1] * dinv, 0.0)
  hw = (jnp.dot(h1a, w_ref[0:64, :], preferred_element_type=jnp.float32)
        + jnp.dot(h1b, w_ref[64:128, :], preferred_element_type=jnp.float32)
        + b_ref[...])
  g = hw * dinv
  g_ref[0] = g[:, :64]
  g_ref[1] = g[:, 64:]


def _tc2(acc1, w2, b2r, degp):
  return pl.pallas_call(
      _tc2_body,
      grid=(NG,),
      in_specs=[
          pl.BlockSpec((2, BLK, 64), lambda i: (0, i, 0)),
          pl.BlockSpec((128, 128), lambda i: (0, 0)),
          pl.BlockSpec((1, 128), lambda i: (0, 0)),
          pl.BlockSpec((2, BLK, 16), lambda i: (0, i, 0)),
      ],
      out_specs=pl.BlockSpec((2, BLK, 64), lambda i: (0, i, 0)),
      out_shape=jax.ShapeDtypeStruct((2, NROW, 64), jnp.float32),
  )(acc1, w2, b2r, degp)


def _tc3_body(trip_ref, dg_ref, p1_ref, pb1_ref, p2_ref, pb2_ref, out_ref):
  def proj(t):
    dg = dg_ref[t, :, 0:1]
    za = trip_ref[t, 0] * dg
    zb = trip_ref[t, 1] * dg
    y = jnp.maximum(
        jnp.dot(za, p1_ref[0:64, :], preferred_element_type=jnp.float32)
        + jnp.dot(zb, p1_ref[64:128, :], preferred_element_type=jnp.float32)
        + pb1_ref[...], 0.0)
    return jnp.dot(y, p2_ref[...],
                   preferred_element_type=jnp.float32) + pb2_ref[...]

  sv = proj(0)
  sa = proj(1)
  sb = proj(2)

  def rnorm(u):
    return jnp.maximum(jnp.sqrt(jnp.sum(u * u, axis=-1, keepdims=True)), 1e-8)

  nv = rnorm(sv)
  pos = jnp.sum(sv * sa, axis=-1, keepdims=True) / (nv * rnorm(sa))
  neg = jnp.sum(sv * sb, axis=-1, keepdims=True) / (nv * rnorm(sb))
  loss = jnp.log(1.0 + jnp.exp((neg - pos) / TAU))
  out_ref[...] = jnp.sum(loss, axis=0, keepdims=True) / B


def _tc3(trip, dg, p1, pb1r, p2, pb2r):
  return pl.pallas_call(
      _tc3_body,
      out_shape=jax.ShapeDtypeStruct((1, 1), jnp.float32),
  )(trip, dg, p1, pb1r, p2, pb2r)


def kernel(x, W1, b1, W2, b2, P1, pb1, P2, pb2, edge_index, v_idx, a_idx,
           b_idx):
  xp = jnp.pad(x, ((0, NROW - N), (0, 0)))
  loops = jnp.arange(N, dtype=jnp.int32)
  padr = N + (jnp.arange(EPAD, dtype=jnp.int32) % (NROW - N))
  srcs = jnp.concatenate([edge_index[0], loops, padr])
  dsts = jnp.concatenate([edge_index[1], loops, padr])
  src2 = jnp.stack([srcs, srcs + NROW]).reshape(2, EROWS, 128)
  dstr = dsts.reshape(EROWS, 128)
  vab = jnp.stack([v_idx, a_idx, b_idx]).reshape(1, 3, 1, B)
  vab2 = jnp.concatenate([vab, vab + NROW])

  degp = _sc_deg(dstr)
  g1, dinv16 = _tc1(xp, W1, b1.reshape(1, 128), degp)
  acc1 = _sc_mp(g1.reshape(2 * NROW, 64), src2, dstr)
  g2 = _tc2(acc1, W2, b2.reshape(1, 128), degp)
  _, trip, dg = _sc_mp2(g2.reshape(2 * NROW, 64), src2, dstr, vab2, dinv16)
  loss = _tc3(trip, dg, P1, pb1.reshape(1, 128), P2, pb2.reshape(1, 128))
  return jnp.reshape(loss, ())


# speedup vs baseline: 24.3024x; 1.0323x over previous
import functools

import jax
import jax.numpy as jnp
from jax import lax
from jax.experimental import pallas as pl
from jax.experimental.pallas import tpu as pltpu, tpu_sc as plsc

N = 10000
E = 320000
D = 128
H = 128
B = 4096
TAU = 0.2

NROW = 10112
ET = E + N
EROWS = 2816
EPAD = EROWS * 128 - ET
NCORE = 2
NSUB = 16
NW = NCORE * NSUB
ROWS_PER_TILE = NROW // NSUB
WBATCH = EROWS // NW
SBATCH = EROWS // NSUB
BLK = 1264
NG = NROW // BLK


def _chunks(total, size):
  out = []
  o = 0
  while o < total:
    n = min(size, total - o)
    out.append((o, n))
    o += n
  return out

_MESH = plsc.VectorSubcoreMesh(
    core_axis_name="c", subcore_axis_name="s", num_cores=NCORE,
    num_subcores=NSUB)
_SC_PARAMS = pltpu.CompilerParams(
    use_tc_tiling_on_sc=False, needs_layout_passes=False)


def _zero_vmem(ref, nrows, ncolgroups):
  z = jnp.zeros((16,), jnp.float32)

  def body(i, _):
    for k in range(ncolgroups):
      ref[i, pl.ds(k * 16, 16)] = z
    return None

  lax.fori_loop(0, nrows, body, None)


@functools.partial(
    pl.kernel,
    out_type=jax.ShapeDtypeStruct((NCORE, NROW, 16), jnp.float32),
    mesh=_MESH,
    compiler_params=_SC_PARAMS,
    scratch_types=[
        pltpu.VMEM((WBATCH, 128), jnp.int32),
        pltpu.VMEM((128, 16), jnp.float32),
        pltpu.VMEM((64, 16), jnp.float32),
        pltpu.VMEM_SHARED((NROW, 16), jnp.float32),
        pltpu.SemaphoreType.DMA,
        pltpu.SemaphoreType.DMA,
        pltpu.SemaphoreType.DMA,
        pltpu.SemaphoreType.DMA,
    ],
)
def _sc_deg(dst_hbm, out_hbm, idx_v, ones_v, zbuf, acc, d0, d1, d2, d3):
  c = lax.axis_index("c")
  s = lax.axis_index("s")
  w = s * NCORE + c
  rows0 = s * ROWS_PER_TILE

  _zero_vmem(zbuf, 64, 1)
  one = jnp.ones((16,), jnp.float32)

  def fill_ones(i, _):
    ones_v[i, :] = one
    return None

  lax.fori_loop(0, 128, fill_ones, None)
  for (o, n) in _chunks(ROWS_PER_TILE, 64):
    pltpu.sync_copy(zbuf.at[pl.ds(0, n)], acc.at[pl.ds(rows0 + o, n)])
  pltpu.sync_copy(dst_hbm.at[pl.ds(w * WBATCH, WBATCH)], idx_v)
  plsc.subcore_barrier()

  sems = (d0, d1, d2, d3)

  def s_start(k, j):
    pltpu.async_copy(ones_v, acc.at[idx_v.at[j]], sems[k], add=True)

  def s_wait(k, j):
    pltpu.make_async_copy(ones_v, acc.at[idx_v.at[j]], sems[k]).wait()

  for k in range(4):
    s_start(k, k)

  def body(jj, _):
    for k in range(4):
      j = 4 * jj + k
      s_wait(k, j - 4)
      s_start(k, j)
    return None

  lax.fori_loop(1, WBATCH // 4, body, None)
  for k in range(4):
    s_wait(k, WBATCH - 4 + k)
  plsc.subcore_barrier()
  for (o, n) in _chunks(ROWS_PER_TILE, 64):
    pltpu.sync_copy(acc.at[pl.ds(rows0 + o, n)], zbuf.at[pl.ds(0, n)])
    pltpu.sync_copy(zbuf.at[pl.ds(0, n)], out_hbm.at[c, pl.ds(rows0 + o, n)])


def _mp_body(c, s, tbl_hbm, src2_hbm, dst_hbm, sidx, didx, rbufs, zbuf, acc,
             sgs, sss):
  rows0 = s * ROWS_PER_TILE
  _zero_vmem(zbuf, 64, 4)
  for (o, n) in _chunks(ROWS_PER_TILE, 64):
    pltpu.sync_copy(zbuf.at[pl.ds(0, n)], acc.at[pl.ds(rows0 + o, n)])
  pltpu.sync_copy(src2_hbm.at[c, pl.ds(s * SBATCH, SBATCH)], sidx)
  pltpu.sync_copy(dst_hbm.at[pl.ds(s * SBATCH, SBATCH)], didx)
  plsc.subcore_barrier()

  def g_start(k, j):
    pltpu.make_async_copy(tbl_hbm.at[sidx.at[j]], rbufs[k], sgs[k]).start()

  def g_wait(k, j):
    pltpu.make_async_copy(tbl_hbm.at[sidx.at[j]], rbufs[k], sgs[k]).wait()

  def s_start(k, j):
    pltpu.async_copy(rbufs[k], acc.at[didx.at[j]], sss[k], add=True)

  def s_wait(k, j):
    pltpu.make_async_copy(rbufs[k], acc.at[didx.at[j]], sss[k]).wait()

  def step(j, k):
    g_wait(k, j)
    s_start(k, j)

  g_start(0, 0)
  g_start(1, 1)
  step(0, 0)
  g_start(2, 2)
  step(1, 1)
  g_start(3, 3)

  def body(jj, _):
    j0 = 4 * jj
    for m in range(4):
      j = j0 + 2 + m
      k = (2 + m) % 4
      step(j, k)
      kf = m % 4
      s_wait(kf, j - 2)
      g_start(kf, j + 2)
    return None

  lax.fori_loop(0, (SBATCH - 4) // 4, body, None)
  for m in range(2):
    j = SBATCH - 2 + m
    k = (2 + m) % 4
    step(j, k)
    s_wait(m, j - 2)
  s_wait(2, SBATCH - 2)
  s_wait(3, SBATCH - 1)
  plsc.subcore_barrier()


_MP_SCRATCH = [
    pltpu.VMEM((SBATCH, 128), jnp.int32),
    pltpu.VMEM((SBATCH, 128), jnp.int32),
    pltpu.VMEM((128, 64), jnp.float32),
    pltpu.VMEM((128, 64), jnp.float32),
    pltpu.VMEM((128, 64), jnp.float32),
    pltpu.VMEM((128, 64), jnp.float32),
    pltpu.VMEM((64, 64), jnp.float32),
    pltpu.VMEM_SHARED((NROW, 64), jnp.float32),
] + [pltpu.SemaphoreType.DMA] * 8


@functools.partial(
    pl.kernel,
    out_type=jax.ShapeDtypeStruct((NCORE, NROW, 64), jnp.float32),
    mesh=_MESH,
    compiler_params=_SC_PARAMS,
    scratch_types=_MP_SCRATCH,
)
def _sc_mp(tbl_hbm, src2_hbm, dst_hbm, out_hbm, sidx, didx, rb0, rb1, rb2,
           rb3, zbuf, acc, sg0, sg1, sg2, sg3, ss0, ss1, ss2, ss3):
  c = lax.axis_index("c")
  s = lax.axis_index("s")
  _mp_body(c, s, tbl_hbm, src2_hbm, dst_hbm, sidx, didx,
           (rb0, rb1, rb2, rb3), zbuf, acc,
           (sg0, sg1, sg2, sg3), (ss0, ss1, ss2, ss3))
  rows0 = s * ROWS_PER_TILE
  for (o, n) in _chunks(ROWS_PER_TILE, 128):
    pltpu.sync_copy(acc.at[pl.ds(rows0 + o, n)], rb0.at[pl.ds(0, n)])
    pltpu.sync_copy(rb0.at[pl.ds(0, n)], out_hbm.at[c, pl.ds(rows0 + o, n)])


@functools.partial(
    pl.kernel,
    out_type=(
        jax.ShapeDtypeStruct((2 * NROW, 64), jnp.float32),
        jax.ShapeDtypeStruct((3, NCORE, B, 64), jnp.float32),
        jax.ShapeDtypeStruct((3, B, 16), jnp.float32),
    ),
    mesh=_MESH,
    compiler_params=_SC_PARAMS,
    scratch_types=_MP_SCRATCH + [
        pltpu.VMEM((3, 256), jnp.int32),
        pltpu.VMEM((256, 16), jnp.float32),
    ],
)
def _sc_mp2(tbl_hbm, src2_hbm, dst_hbm, vab2_hbm, dinv16_hbm,
            emb_hbm, trip_hbm, dg_hbm,
            sidx, didx, rb0, rb1, rb2, rb3, zbuf, acc,
            sg0, sg1, sg2, sg3, ss0, ss1, ss2, ss3,
            tidx, dgbuf):
  c = lax.axis_index("c")
  s = lax.axis_index("s")
  for t in range(3):
    pltpu.sync_copy(vab2_hbm.at[c, t, 0, pl.ds(s * 256, 256)], tidx.at[t])

  _mp_body(c, s, tbl_hbm, src2_hbm, dst_hbm, sidx, didx,
           (rb0, rb1, rb2, rb3), zbuf, acc,
           (sg0, sg1, sg2, sg3), (ss0, ss1, ss2, ss3))

  rows0 = s * ROWS_PER_TILE
  for (o, n) in _chunks(ROWS_PER_TILE, 128):
    pltpu.sync_copy(acc.at[pl.ds(rows0 + o, n)], rb0.at[pl.ds(0, n)])
    pltpu.sync_copy(rb0.at[pl.ds(0, n)],
                    emb_hbm.at[pl.ds(c * NROW + rows0 + o, n)])
  plsc.subcore_barrier()

  for t in range(3):
    for q in range(2):
      pltpu.async_copy(
          emb_hbm.at[tidx.at[t, pl.ds(q * 128, 128)]], rb0, sg0).wait()
      pltpu.sync_copy(
          rb0, trip_hbm.at[t, c, pl.ds(s * 256 + q * 128, 128)])

  @pl.when(c == 0)
  def _():
    for t in range(3):
      pltpu.async_copy(dinv16_hbm.at[tidx.at[t]], dgbuf, sg1).wait()
      pltpu.sync_copy(dgbuf, dg_hbm.at[t, pl.ds(s * 256, 256)])


def _dinv_block(degp_ref, i):
  deg = degp_ref[0, :, 0:1] + degp_ref[1, :, 0:1]
  dinv = lax.rsqrt(jnp.maximum(deg, 1.0))
  row = lax.broadcasted_iota(jnp.int32, (BLK, 1), 0) + i * BLK
  return jnp.where(row < N, dinv, 0.0)


def _tc1_body(x_ref, w_ref, b_ref, degp_ref, g_ref, dinv_ref):
  dinv = _dinv_block(degp_ref, pl.program_id(0))
  hw = jnp.dot(x_ref[...], w_ref[...],
               preferred_element_type=jnp.float32) + b_ref[...]
  g = hw * dinv
  g_ref[0] = g[:, :64]
  g_ref[1] = g[:, 64:]
  dinv_ref[...] = jnp.broadcast_to(dinv, (BLK, 16))


def _tc1(xp, w1, b1r, degp):
  return pl.pallas_call(
      _tc1_body,
      grid=(NG,),
      in_specs=[
          pl.BlockSpec((BLK, 128), lambda i: (i, 0)),
          pl.BlockSpec((128, 128), lambda i: (0, 0)),
          pl.BlockSpec((1, 128), lambda i: (0, 0)),
          pl.BlockSpec((2, BLK, 16), lambda i: (0, i, 0)),
      ],
      out_specs=[
          pl.BlockSpec((2, BLK, 64), lambda i: (0, i, 0)),
          pl.BlockSpec((BLK, 16), lambda i: (i, 0)),
      ],
      out_shape=[
          jax.ShapeDtypeStruct((2, NROW, 64), jnp.float32),
          jax.ShapeDtypeStruct((NROW, 16), jnp.float32),
      ],
  )(xp, w1, b1r, degp)


def _tc2_body(acc_ref, w_ref, b_ref, degp_ref, g_ref):
  dinv = _dinv_block(degp_ref, pl.program_id(0))
  h1a = jnp.maximum(acc_ref[0] * dinv, 0.0)
  h1b = jnp.maximum(acc_ref[1] * dinv, 0.0)
  hw = (jnp.dot(h1a, w_ref[0:64, :], preferred_element_type=jnp.float32)
        + jnp.dot(h1b, w_ref[64:128, :], preferred_element_type=jnp.float32)
        + b_ref[...])
  g = hw * dinv
  g_ref[0] = g[:, :64]
  g_ref[1] = g[:, 64:]


def _tc2(acc1, w2, b2r, degp):
  return pl.pallas_call(
      _tc2_body,
      grid=(NG,),
      in_specs=[
          pl.BlockSpec((2, BLK, 64), lambda i: (0, i, 0)),
          pl.BlockSpec((128, 128), lambda i: (0, 0)),
          pl.BlockSpec((1, 128), lambda i: (0, 0)),
          pl.BlockSpec((2, BLK, 16), lambda i: (0, i, 0)),
      ],
      out_specs=pl.BlockSpec((2, BLK, 64), lambda i: (0, i, 0)),
      out_shape=jax.ShapeDtypeStruct((2, NROW, 64), jnp.float32),
  )(acc1, w2, b2r, degp)


def _tc3_body(trip_ref, dg_ref, p1_ref, pb1_ref, p2_ref, pb2_ref, out_ref):
  def proj(t):
    dg = dg_ref[t, :, 0:1]
    za = trip_ref[t, 0] * dg
    zb = trip_ref[t, 1] * dg
    y = jnp.maximum(
        jnp.dot(za, p1_ref[0:64, :], preferred_element_type=jnp.float32)
        + jnp.dot(zb, p1_ref[64:128, :], preferred_element_type=jnp.float32)
        + pb1_ref[...], 0.0)
    return jnp.dot(y, p2_ref[...],
                   preferred_element_type=jnp.float32) + pb2_ref[...]

  sv = proj(0)
  sa = proj(1)
  sb = proj(2)

  def rnorm(u):
    return jnp.maximum(jnp.sqrt(jnp.sum(u * u, axis=-1, keepdims=True)), 1e-8)

  nv = rnorm(sv)
  pos = jnp.sum(sv * sa, axis=-1, keepdims=True) / (nv * rnorm(sa))
  neg = jnp.sum(sv * sb, axis=-1, keepdims=True) / (nv * rnorm(sb))
  loss = jnp.log(1.0 + jnp.exp((neg - pos) / TAU))
  out_ref[...] = jnp.sum(loss, axis=0, keepdims=True) / B


def _tc3(trip, dg, p1, pb1r, p2, pb2r):
  return pl.pallas_call(
      _tc3_body,
      out_shape=jax.ShapeDtypeStruct((1, 1), jnp.float32),
  )(trip, dg, p1, pb1r, p2, pb2r)


def kernel(x, W1, b1, W2, b2, P1, pb1, P2, pb2, edge_index, v_idx, a_idx,
           b_idx):
  xp = jnp.pad(x, ((0, NROW - N), (0, 0)))
  loops = jnp.arange(N, dtype=jnp.int32)
  padr = N + (jnp.arange(EPAD, dtype=jnp.int32) % (NROW - N))
  srcs = jnp.concatenate([edge_index[0], loops, padr])
  dsts = jnp.concatenate([edge_index[1], loops, padr])
  src2 = jnp.stack([srcs, srcs + NROW]).reshape(2, EROWS, 128)
  dstr = dsts.reshape(EROWS, 128)
  vab = jnp.stack([v_idx, a_idx, b_idx]).reshape(1, 3, 1, B)
  vab2 = jnp.concatenate([vab, vab + NROW])

  degp = _sc_deg(dstr)
  g1, dinv16 = _tc1(xp, W1, b1.reshape(1, 128), degp)
  acc1 = _sc_mp(g1.reshape(2 * NROW, 64), src2, dstr)
  g2 = _tc2(acc1, W2, b2.reshape(1, 128), degp)
  _, trip, dg = _sc_mp2(g2.reshape(2 * NROW, 64), src2, dstr, vab2, dinv16)
  loss = _tc3(trip, dg, P1, pb1.reshape(1, 128), P2, pb2.reshape(1, 128))
  return jnp.reshape(loss, ())


# speedup vs baseline: 25.1398x; 1.0345x over previous
import functools

import jax
import jax.numpy as jnp
from jax import lax
from jax.experimental import pallas as pl
from jax.experimental.pallas import tpu as pltpu, tpu_sc as plsc

N = 10000
E = 320000
D = 128
H = 128
B = 4096
TAU = 0.2

NROW = 10112
ET = E + N
EROWS = 2688
EPAD = EROWS * 128 - ET
NCORE = 2
NSUB = 16
NW = NCORE * NSUB
ROWS_PER_TILE = NROW // NSUB
WBATCH = EROWS // NW
SBATCH = EROWS // NSUB
BLK = 1264
NG = NROW // BLK


def _chunks(total, size):
  out = []
  o = 0
  while o < total:
    n = min(size, total - o)
    out.append((o, n))
    o += n
  return out

_MESH = plsc.VectorSubcoreMesh(
    core_axis_name="c", subcore_axis_name="s", num_cores=NCORE,
    num_subcores=NSUB)
_SC_PARAMS = pltpu.CompilerParams(
    use_tc_tiling_on_sc=False, needs_layout_passes=False)


def _zero_vmem(ref, nrows, ncolgroups):
  z = jnp.zeros((16,), jnp.float32)

  def body(i, _):
    for k in range(ncolgroups):
      ref[i, pl.ds(k * 16, 16)] = z
    return None

  lax.fori_loop(0, nrows, body, None)


@functools.partial(
    pl.kernel,
    out_type=jax.ShapeDtypeStruct((NCORE, NROW, 16), jnp.float32),
    mesh=_MESH,
    compiler_params=_SC_PARAMS,
    scratch_types=[
        pltpu.VMEM((WBATCH, 128), jnp.int32),
        pltpu.VMEM((128, 16), jnp.float32),
        pltpu.VMEM((64, 16), jnp.float32),
        pltpu.VMEM_SHARED((NROW, 16), jnp.float32),
        pltpu.SemaphoreType.DMA,
        pltpu.SemaphoreType.DMA,
        pltpu.SemaphoreType.DMA,
        pltpu.SemaphoreType.DMA,
    ],
)
def _sc_deg(dst_hbm, out_hbm, idx_v, ones_v, zbuf, acc, d0, d1, d2, d3):
  c = lax.axis_index("c")
  s = lax.axis_index("s")
  w = s * NCORE + c
  rows0 = s * ROWS_PER_TILE

  _zero_vmem(zbuf, 64, 1)
  one = jnp.ones((16,), jnp.float32)

  def fill_ones(i, _):
    ones_v[i, :] = one
    return None

  lax.fori_loop(0, 128, fill_ones, None)
  for (o, n) in _chunks(ROWS_PER_TILE, 64):
    pltpu.sync_copy(zbuf.at[pl.ds(0, n)], acc.at[pl.ds(rows0 + o, n)])
  pltpu.sync_copy(dst_hbm.at[pl.ds(w * WBATCH, WBATCH)], idx_v)
  plsc.subcore_barrier()

  sems = (d0, d1, d2, d3)

  def s_start(k, j):
    pltpu.async_copy(ones_v, acc.at[idx_v.at[j]], sems[k], add=True)

  def s_wait(k, j):
    pltpu.make_async_copy(ones_v, acc.at[idx_v.at[j]], sems[k]).wait()

  for k in range(4):
    s_start(k, k)

  def body(jj, _):
    for k in range(4):
      j = 4 * jj + k
      s_wait(k, j - 4)
      s_start(k, j)
    return None

  lax.fori_loop(1, WBATCH // 4, body, None)
  for k in range(4):
    s_wait(k, WBATCH - 4 + k)
  plsc.subcore_barrier()
  for (o, n) in _chunks(ROWS_PER_TILE, 64):
    pltpu.sync_copy(acc.at[pl.ds(rows0 + o, n)], zbuf.at[pl.ds(0, n)])
    pltpu.sync_copy(zbuf.at[pl.ds(0, n)], out_hbm.at[c, pl.ds(rows0 + o, n)])


def _mp_body(c, s, tbl_hbm, src2_hbm, dst_hbm, sidx, didx, rbufs, zbuf, acc,
             sgs, sss):
  rows0 = s * ROWS_PER_TILE
  _zero_vmem(zbuf, 64, 4)
  for (o, n) in _chunks(ROWS_PER_TILE, 64):
    pltpu.sync_copy(zbuf.at[pl.ds(0, n)], acc.at[pl.ds(rows0 + o, n)])
  pltpu.sync_copy(src2_hbm.at[c, pl.ds(s * SBATCH, SBATCH)], sidx)
  pltpu.sync_copy(dst_hbm.at[pl.ds(s * SBATCH, SBATCH)], didx)
  plsc.subcore_barrier()

  def g_start(k, j):
    pltpu.make_async_copy(tbl_hbm.at[sidx.at[j]], rbufs[k], sgs[k]).start()

  def g_wait(k, j):
    pltpu.make_async_copy(tbl_hbm.at[sidx.at[j]], rbufs[k], sgs[k]).wait()

  def s_start(k, j):
    pltpu.async_copy(rbufs[k], acc.at[didx.at[j]], sss[k], add=True)

  def s_wait(k, j):
    pltpu.make_async_copy(rbufs[k], acc.at[didx.at[j]], sss[k]).wait()

  def step(j, k):
    g_wait(k, j)
    s_start(k, j)

  g_start(0, 0)
  g_start(1, 1)
  step(0, 0)
  g_start(2, 2)
  step(1, 1)
  g_start(3, 3)

  def body(jj, _):
    j0 = 4 * jj
    for m in range(4):
      j = j0 + 2 + m
      k = (2 + m) % 4
      step(j, k)
      kf = m % 4
      s_wait(kf, j - 2)
      g_start(kf, j + 2)
    return None

  lax.fori_loop(0, (SBATCH - 4) // 4, body, None)
  for m in range(2):
    j = SBATCH - 2 + m
    k = (2 + m) % 4
    step(j, k)
    s_wait(m, j - 2)
  s_wait(2, SBATCH - 2)
  s_wait(3, SBATCH - 1)
  plsc.subcore_barrier()


_MP_SCRATCH = [
    pltpu.VMEM((SBATCH, 128), jnp.int32),
    pltpu.VMEM((SBATCH, 128), jnp.int32),
    pltpu.VMEM((128, 64), jnp.float32),
    pltpu.VMEM((128, 64), jnp.float32),
    pltpu.VMEM((128, 64), jnp.float32),
    pltpu.VMEM((128, 64), jnp.float32),
    pltpu.VMEM((64, 64), jnp.float32),
    pltpu.VMEM_SHARED((NROW, 64), jnp.float32),
] + [pltpu.SemaphoreType.DMA] * 8


@functools.partial(
    pl.kernel,
    out_type=jax.ShapeDtypeStruct((NCORE, NROW, 64), jnp.float32),
    mesh=_MESH,
    compiler_params=_SC_PARAMS,
    scratch_types=_MP_SCRATCH,
)
def _sc_mp(tbl_hbm, src2_hbm, dst_hbm, out_hbm, sidx, didx, rb0, rb1, rb2,
           rb3, zbuf, acc, sg0, sg1, sg2, sg3, ss0, ss1, ss2, ss3):
  c = lax.axis_index("c")
  s = lax.axis_index("s")
  _mp_body(c, s, tbl_hbm, src2_hbm, dst_hbm, sidx, didx,
           (rb0, rb1, rb2, rb3), zbuf, acc,
           (sg0, sg1, sg2, sg3), (ss0, ss1, ss2, ss3))
  rows0 = s * ROWS_PER_TILE
  for (o, n) in _chunks(ROWS_PER_TILE, 128):
    pltpu.sync_copy(acc.at[pl.ds(rows0 + o, n)], rb0.at[pl.ds(0, n)])
    pltpu.sync_copy(rb0.at[pl.ds(0, n)], out_hbm.at[c, pl.ds(rows0 + o, n)])


@functools.partial(
    pl.kernel,
    out_type=(
        jax.ShapeDtypeStruct((2 * NROW, 64), jnp.float32),
        jax.ShapeDtypeStruct((3, NCORE, B, 64), jnp.float32),
        jax.ShapeDtypeStruct((3, B, 16), jnp.float32),
    ),
    mesh=_MESH,
    compiler_params=_SC_PARAMS,
    scratch_types=_MP_SCRATCH + [
        pltpu.VMEM((3, 256), jnp.int32),
        pltpu.VMEM((256, 16), jnp.float32),
    ],
)
def _sc_mp2(tbl_hbm, src2_hbm, dst_hbm, vab2_hbm, dinv16_hbm,
            emb_hbm, trip_hbm, dg_hbm,
            sidx, didx, rb0, rb1, rb2, rb3, zbuf, acc,
            sg0, sg1, sg2, sg3, ss0, ss1, ss2, ss3,
            tidx, dgbuf):
  c = lax.axis_index("c")
  s = lax.axis_index("s")
  for t in range(3):
    pltpu.sync_copy(vab2_hbm.at[c, t, 0, pl.ds(s * 256, 256)], tidx.at[t])

  _mp_body(c, s, tbl_hbm, src2_hbm, dst_hbm, sidx, didx,
           (rb0, rb1, rb2, rb3), zbuf, acc,
           (sg0, sg1, sg2, sg3), (ss0, ss1, ss2, ss3))

  rows0 = s * ROWS_PER_TILE
  for (o, n) in _chunks(ROWS_PER_TILE, 128):
    pltpu.sync_copy(acc.at[pl.ds(rows0 + o, n)], rb0.at[pl.ds(0, n)])
    pltpu.sync_copy(rb0.at[pl.ds(0, n)],
                    emb_hbm.at[pl.ds(c * NROW + rows0 + o, n)])
  plsc.subcore_barrier()

  for t in range(3):
    for q in range(2):
      pltpu.async_copy(
          emb_hbm.at[tidx.at[t, pl.ds(q * 128, 128)]], rb0, sg0).wait()
      pltpu.sync_copy(
          rb0, trip_hbm.at[t, c, pl.ds(s * 256 + q * 128, 128)])

  @pl.when(c == 0)
  def _():
    for t in range(3):
      pltpu.async_copy(dinv16_hbm.at[tidx.at[t]], dgbuf, sg1).wait()
      pltpu.sync_copy(dgbuf, dg_hbm.at[t, pl.ds(s * 256, 256)])


def _dinv_block(degp_ref, i):
  deg = degp_ref[0, :, 0:1] + degp_ref[1, :, 0:1]
  dinv = lax.rsqrt(jnp.maximum(deg, 1.0))
  row = lax.broadcasted_iota(jnp.int32, (BLK, 1), 0) + i * BLK
  return jnp.where(row < N, dinv, 0.0)


def _tc1_body(x_ref, w_ref, b_ref, degp_ref, g_ref, dinv_ref):
  dinv = _dinv_block(degp_ref, pl.program_id(0))
  hw = jnp.dot(x_ref[...], w_ref[...],
               preferred_element_type=jnp.float32) + b_ref[...]
  g = hw * dinv
  g_ref[0] = g[:, :64]
  g_ref[1] = g[:, 64:]
  dinv_ref[...] = jnp.broadcast_to(dinv, (BLK, 16))


def _tc1(xp, w1, b1r, degp):
  return pl.pallas_call(
      _tc1_body,
      grid=(NG,),
      in_specs=[
          pl.BlockSpec((BLK, 128), lambda i: (i, 0)),
          pl.BlockSpec((128, 128), lambda i: (0, 0)),
          pl.BlockSpec((1, 128), lambda i: (0, 0)),
          pl.BlockSpec((2, BLK, 16), lambda i: (0, i, 0)),
      ],
      out_specs=[
          pl.BlockSpec((2, BLK, 64), lambda i: (0, i, 0)),
          pl.BlockSpec((BLK, 16), lambda i: (i, 0)),
      ],
      out_shape=[
          jax.ShapeDtypeStruct((2, NROW, 64), jnp.float32),
          jax.ShapeDtypeStruct((NROW, 16), jnp.float32),
      ],
  )(xp, w1, b1r, degp)


def _tc2_body(acc_ref, w_ref, b_ref, degp_ref, g_ref):
  dinv = _dinv_block(degp_ref, pl.program_id(0))
  h1a = jnp.maximum(acc_ref[0] * dinv, 0.0)
  h1b = jnp.maximum(acc_ref[1] * dinv, 0.0)
  hw = (jnp.dot(h1a, w_ref[0:64, :], preferred_element_type=jnp.float32)
        + jnp.dot(h1b, w_ref[64:128, :], preferred_element_type=jnp.float32)
        + b_ref[...])
  g = hw * dinv
  g_ref[0] = g[:, :64]
  g_ref[1] = g[:, 64:]


def _tc2(acc1, w2, b2r, degp):
  return pl.pallas_call(
      _tc2_body,
      grid=(NG,),
      in_specs=[
          pl.BlockSpec((2, BLK, 64), lambda i: (0, i, 0)),
          pl.BlockSpec((128, 128), lambda i: (0, 0)),
          pl.BlockSpec((1, 128), lambda i: (0, 0)),
          pl.BlockSpec((2, BLK, 16), lambda i: (0, i, 0)),
      ],
      out_specs=pl.BlockSpec((2, BLK, 64), lambda i: (0, i, 0)),
      out_shape=jax.ShapeDtypeStruct((2, NROW, 64), jnp.float32),
  )(acc1, w2, b2r, degp)


def _tc3_body(trip_ref, dg_ref, p1_ref, pb1_ref, p2_ref, pb2_ref, out_ref):
  def proj(t):
    dg = dg_ref[t, :, 0:1]
    za = trip_ref[t, 0] * dg
    zb = trip_ref[t, 1] * dg
    y = jnp.maximum(
        jnp.dot(za, p1_ref[0:64, :], preferred_element_type=jnp.float32)
        + jnp.dot(zb, p1_ref[64:128, :], preferred_element_type=jnp.float32)
        + pb1_ref[...], 0.0)
    return jnp.dot(y, p2_ref[...],
                   preferred_element_type=jnp.float32) + pb2_ref[...]

  sv = proj(0)
  sa = proj(1)
  sb = proj(2)

  def rnorm(u):
    return jnp.maximum(jnp.sqrt(jnp.sum(u * u, axis=-1, keepdims=True)), 1e-8)

  nv = rnorm(sv)
  pos = jnp.sum(sv * sa, axis=-1, keepdims=True) / (nv * rnorm(sa))
  neg = jnp.sum(sv * sb, axis=-1, keepdims=True) / (nv * rnorm(sb))
  loss = jnp.log(1.0 + jnp.exp((neg - pos) / TAU))
  out_ref[...] = jnp.sum(loss, axis=0, keepdims=True) / B


def _tc3(trip, dg, p1, pb1r, p2, pb2r):
  return pl.pallas_call(
      _tc3_body,
      out_shape=jax.ShapeDtypeStruct((1, 1), jnp.float32),
  )(trip, dg, p1, pb1r, p2, pb2r)


def kernel(x, W1, b1, W2, b2, P1, pb1, P2, pb2, edge_index, v_idx, a_idx,
           b_idx):
  xp = jnp.pad(x, ((0, NROW - N), (0, 0)))
  loops = jnp.arange(N, dtype=jnp.int32)
  padr = N + (jnp.arange(EPAD, dtype=jnp.int32) % (NROW - N))
  srcs = jnp.concatenate([edge_index[0], loops, padr])
  dsts = jnp.concatenate([edge_index[1], loops, padr])
  src2 = jnp.stack([srcs, srcs + NROW]).reshape(2, EROWS, 128)
  dstr = dsts.reshape(EROWS, 128)
  vab = jnp.stack([v_idx, a_idx, b_idx]).reshape(1, 3, 1, B)
  vab2 = jnp.concatenate([vab, vab + NROW])

  degp = _sc_deg(dstr)
  g1, dinv16 = _tc1(xp, W1, b1.reshape(1, 128), degp)
  acc1 = _sc_mp(g1.reshape(2 * NROW, 64), src2, dstr)
  g2 = _tc2(acc1, W2, b2.reshape(1, 128), degp)
  _, trip, dg = _sc_mp2(g2.reshape(2 * NROW, 64), src2, dstr, vab2, dinv16)
  loss = _tc3(trip, dg, P1, pb1.reshape(1, 128), P2, pb2.reshape(1, 128))
  return jnp.reshape(loss, ())


# speedup vs baseline: 25.5054x; 1.0145x over previous
import functools

import jax
import jax.numpy as jnp
from jax import lax
from jax.experimental import pallas as pl
from jax.experimental.pallas import tpu as pltpu, tpu_sc as plsc

N = 10000
E = 320000
D = 128
H = 128
B = 4096
TAU = 0.2

NROW = 10112
ET = E + N
EROWS = 2688
EPAD = EROWS * 128 - ET
NCORE = 2
NSUB = 16
NW = NCORE * NSUB
ROWS_PER_TILE = NROW // NSUB
WBATCH = EROWS // NW
SBATCH = EROWS // NSUB
BLK = 1264
NG = NROW // BLK


def _chunks(total, size):
  out = []
  o = 0
  while o < total:
    n = min(size, total - o)
    out.append((o, n))
    o += n
  return out

_MESH = plsc.VectorSubcoreMesh(
    core_axis_name="c", subcore_axis_name="s", num_cores=NCORE,
    num_subcores=NSUB)
_SC_PARAMS = pltpu.CompilerParams(
    use_tc_tiling_on_sc=False, needs_layout_passes=False)


def _zero_vmem(ref, nrows, ncolgroups):
  z = jnp.zeros((16,), jnp.float32)

  def body(i, _):
    for k in range(ncolgroups):
      ref[i, pl.ds(k * 16, 16)] = z
    return None

  lax.fori_loop(0, nrows, body, None)


@functools.partial(
    pl.kernel,
    out_type=jax.ShapeDtypeStruct((NCORE, NROW, 16), jnp.float32),
    mesh=_MESH,
    compiler_params=_SC_PARAMS,
    scratch_types=[
        pltpu.VMEM((WBATCH, 128), jnp.int32),
        pltpu.VMEM((128, 16), jnp.float32),
        pltpu.VMEM((64, 16), jnp.float32),
        pltpu.VMEM_SHARED((NROW, 16), jnp.float32),
        pltpu.SemaphoreType.DMA,
        pltpu.SemaphoreType.DMA,
        pltpu.SemaphoreType.DMA,
        pltpu.SemaphoreType.DMA,
    ],
)
def _sc_deg(dst_hbm, out_hbm, idx_v, ones_v, zbuf, acc, d0, d1, d2, d3):
  c = lax.axis_index("c")
  s = lax.axis_index("s")
  w = s * NCORE + c
  rows0 = s * ROWS_PER_TILE

  _zero_vmem(zbuf, 64, 1)
  one = jnp.ones((16,), jnp.float32)

  def fill_ones(i, _):
    ones_v[i, :] = one
    return None

  lax.fori_loop(0, 128, fill_ones, None)
  for (o, n) in _chunks(ROWS_PER_TILE, 64):
    pltpu.sync_copy(zbuf.at[pl.ds(0, n)], acc.at[pl.ds(rows0 + o, n)])
  pltpu.sync_copy(dst_hbm.at[pl.ds(w * WBATCH, WBATCH)], idx_v)
  plsc.subcore_barrier()

  sems = (d0, d1, d2, d3)

  def s_start(k, j):
    pltpu.async_copy(ones_v, acc.at[idx_v.at[j]], sems[k], add=True)

  def s_wait(k, j):
    pltpu.make_async_copy(ones_v, acc.at[idx_v.at[j]], sems[k]).wait()

  for k in range(4):
    s_start(k, k)

  def body(jj, _):
    for k in range(4):
      j = 4 * jj + k
      s_wait(k, j - 4)
      s_start(k, j)
    return None

  lax.fori_loop(1, WBATCH // 4, body, None)
  for k in range(4):
    s_wait(k, WBATCH - 4 + k)
  plsc.subcore_barrier()
  for (o, n) in _chunks(ROWS_PER_TILE, 64):
    pltpu.sync_copy(acc.at[pl.ds(rows0 + o, n)], zbuf.at[pl.ds(0, n)])
    pltpu.sync_copy(zbuf.at[pl.ds(0, n)], out_hbm.at[c, pl.ds(rows0 + o, n)])


def _mp_body(c, s, tbl_hbm, src2_hbm, dst_hbm, sidx, didx, rbufs, zbuf, acc,
             sgs, sss):
  rows0 = s * ROWS_PER_TILE
  _zero_vmem(zbuf, 64, 4)
  for (o, n) in _chunks(ROWS_PER_TILE, 64):
    pltpu.sync_copy(zbuf.at[pl.ds(0, n)], acc.at[pl.ds(rows0 + o, n)])
  pltpu.sync_copy(src2_hbm.at[c, pl.ds(s * SBATCH, SBATCH)], sidx)
  pltpu.sync_copy(dst_hbm.at[pl.ds(s * SBATCH, SBATCH)], didx)
  plsc.subcore_barrier()

  def g_start(k, j):
    pltpu.make_async_copy(tbl_hbm.at[sidx.at[j]], rbufs[k], sgs[k]).start()

  def g_wait(k, j):
    pltpu.make_async_copy(tbl_hbm.at[sidx.at[j]], rbufs[k], sgs[k]).wait()

  def s_start(k, j):
    pltpu.async_copy(rbufs[k], acc.at[didx.at[j]], sss[k], add=True)

  def s_wait(k, j):
    pltpu.make_async_copy(rbufs[k], acc.at[didx.at[j]], sss[k]).wait()

  def step(j, k):
    g_wait(k, j)
    s_start(k, j)

  g_start(0, 0)
  g_start(1, 1)
  step(0, 0)
  g_start(2, 2)
  step(1, 1)
  g_start(3, 3)

  def body(jj, _):
    j0 = 4 * jj
    for m in range(4):
      j = j0 + 2 + m
      k = (2 + m) % 4
      step(j, k)
      kf = m % 4
      s_wait(kf, j - 2)
      g_start(kf, j + 2)
    return None

  lax.fori_loop(0, (SBATCH - 4) // 4, body, None)
  for m in range(2):
    j = SBATCH - 2 + m
    k = (2 + m) % 4
    step(j, k)
    s_wait(m, j - 2)
  s_wait(2, SBATCH - 2)
  s_wait(3, SBATCH - 1)
  plsc.subcore_barrier()


_MP_SCRATCH = [
    pltpu.VMEM((SBATCH, 128), jnp.int32),
    pltpu.VMEM((SBATCH, 128), jnp.int32),
    pltpu.VMEM((128, 64), jnp.float32),
    pltpu.VMEM((128, 64), jnp.float32),
    pltpu.VMEM((128, 64), jnp.float32),
    pltpu.VMEM((128, 64), jnp.float32),
    pltpu.VMEM((64, 64), jnp.float32),
    pltpu.VMEM_SHARED((NROW, 64), jnp.float32),
] + [pltpu.SemaphoreType.DMA] * 8


@functools.partial(
    pl.kernel,
    out_type=jax.ShapeDtypeStruct((NCORE, NROW, 64), jnp.float32),
    mesh=_MESH,
    compiler_params=_SC_PARAMS,
    scratch_types=_MP_SCRATCH,
)
def _sc_mp(tbl_hbm, src2_hbm, dst_hbm, out_hbm, sidx, didx, rb0, rb1, rb2,
           rb3, zbuf, acc, sg0, sg1, sg2, sg3, ss0, ss1, ss2, ss3):
  c = lax.axis_index("c")
  s = lax.axis_index("s")
  _mp_body(c, s, tbl_hbm, src2_hbm, dst_hbm, sidx, didx,
           (rb0, rb1, rb2, rb3), zbuf, acc,
           (sg0, sg1, sg2, sg3), (ss0, ss1, ss2, ss3))
  rows0 = s * ROWS_PER_TILE
  for (o, n) in _chunks(ROWS_PER_TILE, 128):
    pltpu.sync_copy(acc.at[pl.ds(rows0 + o, n)], rb0.at[pl.ds(0, n)])
    pltpu.sync_copy(rb0.at[pl.ds(0, n)], out_hbm.at[c, pl.ds(rows0 + o, n)])


@functools.partial(
    pl.kernel,
    out_type=(
        jax.ShapeDtypeStruct((2 * NROW, 64), jnp.float32),
        jax.ShapeDtypeStruct((3, NCORE, B, 64), jnp.float32),
        jax.ShapeDtypeStruct((3, B, 16), jnp.float32),
    ),
    mesh=_MESH,
    compiler_params=_SC_PARAMS,
    scratch_types=_MP_SCRATCH + [
        pltpu.VMEM((3, 256), jnp.int32),
        pltpu.VMEM((256, 16), jnp.float32),
    ],
)
def _sc_mp2(tbl_hbm, src2_hbm, dst_hbm, vab2_hbm, dinv16_hbm,
            emb_hbm, trip_hbm, dg_hbm,
            sidx, didx, rb0, rb1, rb2, rb3, zbuf, acc,
            sg0, sg1, sg2, sg3, ss0, ss1, ss2, ss3,
            tidx, dgbuf):
  c = lax.axis_index("c")
  s = lax.axis_index("s")
  for t in range(3):
    pltpu.sync_copy(vab2_hbm.at[c, t, 0, pl.ds(s * 256, 256)], tidx.at[t])

  _mp_body(c, s, tbl_hbm, src2_hbm, dst_hbm, sidx, didx,
           (rb0, rb1, rb2, rb3), zbuf, acc,
           (sg0, sg1, sg2, sg3), (ss0, ss1, ss2, ss3))

  rows0 = s * ROWS_PER_TILE
  for (o, n) in _chunks(ROWS_PER_TILE, 128):
    pltpu.sync_copy(acc.at[pl.ds(rows0 + o, n)], rb0.at[pl.ds(0, n)])
    pltpu.sync_copy(rb0.at[pl.ds(0, n)],
                    emb_hbm.at[pl.ds(c * NROW + rows0 + o, n)])
  plsc.subcore_barrier()

  tbufs = (rb0, rb1)
  tsems = (sg0, sg1)

  def tg_start(i):
    t, q = divmod(i, 2)
    pltpu.make_async_copy(emb_hbm.at[tidx.at[t, pl.ds(q * 128, 128)]],
                          tbufs[i % 2], tsems[i % 2]).start()

  def tg_finish(i):
    t, q = divmod(i, 2)
    pltpu.make_async_copy(emb_hbm.at[tidx.at[t, pl.ds(q * 128, 128)]],
                          tbufs[i % 2], tsems[i % 2]).wait()
    pltpu.sync_copy(tbufs[i % 2],
                    trip_hbm.at[t, c, pl.ds(s * 256 + q * 128, 128)])

  def dg_start(t):
    pltpu.make_async_copy(dinv16_hbm.at[tidx.at[t]], dgbuf, sg2).start()

  def dg_finish(t):
    pltpu.make_async_copy(dinv16_hbm.at[tidx.at[t]], dgbuf, sg2).wait()
    pltpu.sync_copy(dgbuf, dg_hbm.at[t, pl.ds(s * 256, 256)])

  tg_start(0)
  tg_start(1)

  @pl.when(c == 0)
  def _():
    dg_start(0)

  for i in range(6):
    tg_finish(i)
    if i + 2 < 6:
      tg_start(i + 2)
    if i % 2 == 1:
      t = i // 2

      @pl.when(c == 0)
      def _():
        dg_finish(t)
        if t < 2:
          dg_start(t + 1)


def _dinv_block(degp_ref, i):
  deg = degp_ref[0, :, 0:1] + degp_ref[1, :, 0:1]
  dinv = lax.rsqrt(jnp.maximum(deg, 1.0))
  row = lax.broadcasted_iota(jnp.int32, (BLK, 1), 0) + i * BLK
  return jnp.where(row < N, dinv, 0.0)


def _tc1_body(x_ref, w_ref, b_ref, degp_ref, g_ref, dinv_ref):
  dinv = _dinv_block(degp_ref, pl.program_id(0))
  hw = jnp.dot(x_ref[...], w_ref[...],
               preferred_element_type=jnp.float32) + b_ref[...]
  g = hw * dinv
  g_ref[0] = g[:, :64]
  g_ref[1] = g[:, 64:]
  dinv_ref[...] = jnp.broadcast_to(dinv, (BLK, 16))


def _tc1(xp, w1, b1r, degp):
  return pl.pallas_call(
      _tc1_body,
      grid=(NG,),
      in_specs=[
          pl.BlockSpec((BLK, 128), lambda i: (i, 0)),
          pl.BlockSpec((128, 128), lambda i: (0, 0)),
          pl.BlockSpec((1, 128), lambda i: (0, 0)),
          pl.BlockSpec((2, BLK, 16), lambda i: (0, i, 0)),
      ],
      out_specs=[
          pl.BlockSpec((2, BLK, 64), lambda i: (0, i, 0)),
          pl.BlockSpec((BLK, 16), lambda i: (i, 0)),
      ],
      out_shape=[
          jax.ShapeDtypeStruct((2, NROW, 64), jnp.float32),
          jax.ShapeDtypeStruct((NROW, 16), jnp.float32),
      ],
  )(xp, w1, b1r, degp)


def _tc2_body(acc_ref, w_ref, b_ref, degp_ref, g_ref):
  dinv = _dinv_block(degp_ref, pl.program_id(0))
  h1a = jnp.maximum(acc_ref[0] * dinv, 0.0)
  h1b = jnp.maximum(acc_ref[1] * dinv, 0.0)
  hw = (jnp.dot(h1a, w_ref[0:64, :], preferred_element_type=jnp.float32)
        + jnp.dot(h1b, w_ref[64:128, :], preferred_element_type=jnp.float32)
        + b_ref[...])
  g = hw * dinv
  g_ref[0] = g[:, :64]
  g_ref[1] = g[:, 64:]


def _tc2(acc1, w2, b2r, degp):
  return pl.pallas_call(
      _tc2_body,
      grid=(NG,),
      in_specs=[
          pl.BlockSpec((2, BLK, 64), lambda i: (0, i, 0)),
          pl.BlockSpec((128, 128), lambda i: (0, 0)),
          pl.BlockSpec((1, 128), lambda i: (0, 0)),
          pl.BlockSpec((2, BLK, 16), lambda i: (0, i, 0)),
      ],
      out_specs=pl.BlockSpec((2, BLK, 64), lambda i: (0, i, 0)),
      out_shape=jax.ShapeDtypeStruct((2, NROW, 64), jnp.float32),
  )(acc1, w2, b2r, degp)


def _tc3_body(trip_ref, dg_ref, p1_ref, pb1_ref, p2_ref, pb2_ref, out_ref):
  def proj(t):
    dg = dg_ref[t, :, 0:1]
    za = trip_ref[t, 0] * dg
    zb = trip_ref[t, 1] * dg
    y = jnp.maximum(
        jnp.dot(za, p1_ref[0:64, :], preferred_element_type=jnp.float32)
        + jnp.dot(zb, p1_ref[64:128, :], preferred_element_type=jnp.float32)
        + pb1_ref[...], 0.0)
    return jnp.dot(y, p2_ref[...],
                   preferred_element_type=jnp.float32) + pb2_ref[...]

  sv = proj(0)
  sa = proj(1)
  sb = proj(2)

  def rnorm(u):
    return jnp.maximum(jnp.sqrt(jnp.sum(u * u, axis=-1, keepdims=True)), 1e-8)

  nv = rnorm(sv)
  pos = jnp.sum(sv * sa, axis=-1, keepdims=True) / (nv * rnorm(sa))
  neg = jnp.sum(sv * sb, axis=-1, keepdims=True) / (nv * rnorm(sb))
  loss = jnp.log(1.0 + jnp.exp((neg - pos) / TAU))
  out_ref[...] = jnp.sum(loss, axis=0, keepdims=True) / B


def _tc3(trip, dg, p1, pb1r, p2, pb2r):
  return pl.pallas_call(
      _tc3_body,
      out_shape=jax.ShapeDtypeStruct((1, 1), jnp.float32),
  )(trip, dg, p1, pb1r, p2, pb2r)


def kernel(x, W1, b1, W2, b2, P1, pb1, P2, pb2, edge_index, v_idx, a_idx,
           b_idx):
  xp = jnp.pad(x, ((0, NROW - N), (0, 0)))
  loops = jnp.arange(N, dtype=jnp.int32)
  padr = N + (jnp.arange(EPAD, dtype=jnp.int32) % (NROW - N))
  srcs = jnp.concatenate([edge_index[0], loops, padr])
  dsts = jnp.concatenate([edge_index[1], loops, padr])
  src2 = jnp.stack([srcs, srcs + NROW]).reshape(2, EROWS, 128)
  dstr = dsts.reshape(EROWS, 128)
  vab = jnp.stack([v_idx, a_idx, b_idx]).reshape(1, 3, 1, B)
  vab2 = jnp.concatenate([vab, vab + NROW])

  degp = _sc_deg(dstr)
  g1, dinv16 = _tc1(xp, W1, b1.reshape(1, 128), degp)
  acc1 = _sc_mp(g1.reshape(2 * NROW, 64), src2, dstr)
  g2 = _tc2(acc1, W2, b2.reshape(1, 128), degp)
  _, trip, dg = _sc_mp2(g2.reshape(2 * NROW, 64), src2, dstr, vab2, dinv16)
  loss = _tc3(trip, dg, P1, pb1.reshape(1, 128), P2, pb2.reshape(1, 128))
  return jnp.reshape(loss, ())
